# baseline (device time: 14121 ns/iter reference)
import jax
import jax.numpy as jnp
from jax import lax
from jax.experimental import pallas as pl
from jax.experimental.pallas import tpu as pltpu

N_DEV = 16
B = 2
S = 128
HQ = 4
DH = 64
HD = HQ * DH
NEG = -1e9


def kernel(x, Wq, K_ext, V_ext, Wo):
    d_model = x.shape[-1]

    def body(x_ref, wq_ref, k_ref, v_ref, wo_ref, out_ref,
             ksend, vsend, kbuf, vbuf, send_sems, recv_sems):
        my = lax.axis_index("i")
        left = my - 1
        right = my + 1
        has_left = my > 0
        has_right = my < N_DEV - 1

        @pl.when(jnp.logical_not(has_left))
        def _():
            kbuf[0] = jnp.zeros((B, HQ, DH, S), jnp.bfloat16)
            vbuf[0] = jnp.zeros((B, HQ, DH, S), jnp.bfloat16)

        @pl.when(jnp.logical_not(has_right))
        def _():
            kbuf[1] = jnp.zeros((B, HQ, DH, S), jnp.bfloat16)
            vbuf[1] = jnp.zeros((B, HQ, DH, S), jnp.bfloat16)

        barrier_sem = pltpu.get_barrier_semaphore()

        @pl.when(has_left)
        def _():
            pl.semaphore_signal(barrier_sem, inc=1, device_id=(left,),
                                device_id_type=pl.DeviceIdType.MESH)

        @pl.when(has_right)
        def _():
            pl.semaphore_signal(barrier_sem, inc=1, device_id=(right,),
                                device_id_type=pl.DeviceIdType.MESH)

        ksend[...] = k_ref[...].astype(jnp.bfloat16)
        vsend[...] = v_ref[...].astype(jnp.bfloat16)

        n_nbr = has_left.astype(jnp.int32) + has_right.astype(jnp.int32)
        pl.semaphore_wait(barrier_sem, n_nbr)

        def sem_idx(kind, b, slot):
            return (kind * B + b) * 2 + slot

        def halo_rdma(kind, src, buf, b, slot, nbr):
            i = sem_idx(kind, b, slot)
            return pltpu.make_async_remote_copy(
                src_ref=src.at[b],
                dst_ref=buf.at[slot, b],
                send_sem=send_sems.at[i],
                recv_sem=recv_sems.at[i],
                device_id=(nbr,),
                device_id_type=pl.DeviceIdType.MESH,
            )

        for b in range(B):
            @pl.when(has_right)
            def _(b=b):
                halo_rdma(0, ksend, kbuf, b, 0, right).start()

            @pl.when(has_left)
            def _(b=b):
                halo_rdma(0, ksend, kbuf, b, 1, left).start()

            @pl.when(has_right)
            def _(b=b):
                halo_rdma(1, vsend, vbuf, b, 0, right).start()

            @pl.when(has_left)
            def _(b=b):
                halo_rdma(1, vsend, vbuf, b, 1, left).start()

        x2 = x_ref[...].reshape(B * S, x_ref.shape[-1])
        q2 = jnp.dot(x2, wq_ref[...],
                     preferred_element_type=jnp.float32) * 0.125

        qi = lax.broadcasted_iota(jnp.int32, (S, S), 0)
        kj = lax.broadcasted_iota(jnp.int32, (S, S), 1)
        full_mask = jnp.concatenate(
            [jnp.logical_and(qi <= kj, has_left),
             jnp.ones((S, S), jnp.bool_),
             jnp.logical_and(qi >= kj, has_right)], axis=1)

        dn_qk = (((1,), (0,)), ((), ()))
        dn_vw = (((1,), (1,)), ((), ()))
        dn_out = (((0,), (0,)), ((), ()))
        for b in range(B):
            @pl.when(has_left)
            def _(b=b):
                halo_rdma(0, ksend, kbuf, b, 0, left).wait_recv()

            @pl.when(has_right)
            def _(b=b):
                halo_rdma(0, ksend, kbuf, b, 1, right).wait_recv()

            w_heads = []
            for h in range(HQ):
                k_t = jnp.concatenate(
                    [kbuf[0, b, h], ksend[b, h], kbuf[1, b, h]],
                    axis=1)
                q = q2[b * S:(b + 1) * S,
                       h * DH:(h + 1) * DH].astype(jnp.bfloat16)
                s = lax.dot_general(q, k_t, dn_qk,
                                    preferred_element_type=jnp.float32)
                w = jnp.where(full_mask, jnp.exp(s), 0.0)
                w = w / jnp.sum(w, axis=1, keepdims=True)
                w_heads.append(w.astype(jnp.bfloat16))

            @pl.when(has_left)
            def _(b=b):
                halo_rdma(1, vsend, vbuf, b, 0, left).wait_recv()

            @pl.when(has_right)
            def _(b=b):
                halo_rdma(1, vsend, vbuf, b, 1, right).wait_recv()

            ctx_t = jnp.concatenate(
                [lax.dot_general(
                    jnp.concatenate([vbuf[0, b, h], vsend[b, h],
                                     vbuf[1, b, h]], axis=1),
                    w_heads[h], dn_vw,
                    preferred_element_type=jnp.float32)
                 for h in range(HQ)], axis=0)
            out_ref[b] = lax.dot_general(ctx_t, wo_ref[...], dn_out,
                                         preferred_element_type=jnp.float32)

        for b in range(B):
            @pl.when(has_right)
            def _(b=b):
                halo_rdma(0, ksend, kbuf, b, 0, right).wait_send()
                halo_rdma(1, vsend, vbuf, b, 0, right).wait_send()

            @pl.when(has_left)
            def _(b=b):
                halo_rdma(0, ksend, kbuf, b, 1, left).wait_send()
                halo_rdma(1, vsend, vbuf, b, 1, left).wait_send()

    k_t = jnp.transpose(K_ext, (0, 2, 3, 1))
    v_t = jnp.transpose(V_ext, (0, 2, 3, 1))

    return pl.pallas_call(
        body,
        out_shape=jax.ShapeDtypeStruct((B, S, d_model), jnp.float32),
        in_specs=[pl.BlockSpec(memory_space=pltpu.VMEM)] * 5,
        out_specs=pl.BlockSpec(memory_space=pltpu.VMEM),
        scratch_shapes=[
            pltpu.VMEM((B, HQ, DH, S), jnp.bfloat16),
            pltpu.VMEM((B, HQ, DH, S), jnp.bfloat16),
            pltpu.VMEM((2, B, HQ, DH, S), jnp.bfloat16),
            pltpu.VMEM((2, B, HQ, DH, S), jnp.bfloat16),
            pltpu.SemaphoreType.DMA((8,)),
            pltpu.SemaphoreType.DMA((8,)),
        ],
        compiler_params=pltpu.CompilerParams(collective_id=0),
    )(x, Wq, k_t, v_t, Wo)


# device time: 11600 ns/iter; 1.2173x vs baseline; 1.2173x over previous
import jax
import jax.numpy as jnp
from jax import lax
from jax.experimental import pallas as pl
from jax.experimental.pallas import tpu as pltpu

N_DEV = 16
B = 2
S = 128
HQ = 4
DH = 64
HD = HQ * DH
NEG = -1e9


def kernel(x, Wq, K_ext, V_ext, Wo):
    d_model = x.shape[-1]

    def body(x_ref, wq_ref, k_ref, v_ref, wo_ref, out_ref,
             ksend, vsend, kbuf, vbuf, out_vmem,
             send_sems, recv_sems, out_sems):
        my = lax.axis_index("i")
        left = my - 1
        right = my + 1
        has_left = my > 0
        has_right = my < N_DEV - 1

        @pl.when(jnp.logical_not(has_left))
        def _():
            kbuf[0] = jnp.zeros((B, S, HD), jnp.bfloat16)
            vbuf[0] = jnp.zeros((B, S, HD), jnp.bfloat16)

        @pl.when(jnp.logical_not(has_right))
        def _():
            kbuf[1] = jnp.zeros((B, S, HD), jnp.bfloat16)
            vbuf[1] = jnp.zeros((B, S, HD), jnp.bfloat16)

        barrier_sem = pltpu.get_barrier_semaphore()

        @pl.when(has_left)
        def _():
            pl.semaphore_signal(barrier_sem, inc=1, device_id=(left,),
                                device_id_type=pl.DeviceIdType.MESH)

        @pl.when(has_right)
        def _():
            pl.semaphore_signal(barrier_sem, inc=1, device_id=(right,),
                                device_id_type=pl.DeviceIdType.MESH)

        for h in range(HQ):
            hs = slice(h * DH, (h + 1) * DH)
            ksend[:, :, hs] = k_ref[:, :, h, :].astype(jnp.bfloat16)
            vsend[:, :, hs] = v_ref[:, :, h, :].astype(jnp.bfloat16)

        n_nbr = has_left.astype(jnp.int32) + has_right.astype(jnp.int32)
        pl.semaphore_wait(barrier_sem, n_nbr)

        def sem_idx(kind, b, slot):
            return (kind * B + b) * 2 + slot

        def halo_rdma(kind, src, buf, b, slot, nbr):
            i = sem_idx(kind, b, slot)
            return pltpu.make_async_remote_copy(
                src_ref=src.at[b],
                dst_ref=buf.at[slot, b],
                send_sem=send_sems.at[i],
                recv_sem=recv_sems.at[i],
                device_id=(nbr,),
                device_id_type=pl.DeviceIdType.MESH,
            )

        for b in range(B):
            @pl.when(has_right)
            def _(b=b):
                halo_rdma(0, ksend, kbuf, b, 0, right).start()

            @pl.when(has_left)
            def _(b=b):
                halo_rdma(0, ksend, kbuf, b, 1, left).start()

            @pl.when(has_right)
            def _(b=b):
                halo_rdma(1, vsend, vbuf, b, 0, right).start()

            @pl.when(has_left)
            def _(b=b):
                halo_rdma(1, vsend, vbuf, b, 1, left).start()

        x2 = x_ref[...].reshape(B * S, x_ref.shape[-1])
        q2 = jnp.dot(x2, wq_ref[...],
                     preferred_element_type=jnp.float32) * 0.125

        qi = lax.broadcasted_iota(jnp.int32, (S, S), 0)
        kj = lax.broadcasted_iota(jnp.int32, (S, S), 1)
        full_mask = jnp.concatenate(
            [jnp.logical_and(qi <= kj, has_left),
             jnp.ones((S, S), jnp.bool_),
             jnp.logical_and(qi >= kj, has_right)], axis=1)

        dn = (((1,), (1,)), ((), ()))
        for b in range(B):
            @pl.when(has_left)
            def _(b=b):
                halo_rdma(0, ksend, kbuf, b, 0, left).wait_recv()

            @pl.when(has_right)
            def _(b=b):
                halo_rdma(0, ksend, kbuf, b, 1, right).wait_recv()

            k_full = jnp.concatenate(
                [kbuf[0, b], ksend[b], kbuf[1, b]], axis=0)
            w_heads = []
            for h in range(HQ):
                hs = slice(h * DH, (h + 1) * DH)
                q = q2[b * S:(b + 1) * S, hs].astype(jnp.bfloat16)
                s = lax.dot_general(q, k_full[:, hs], dn,
                                    preferred_element_type=jnp.float32)
                w = jnp.where(full_mask, jnp.exp(s), 0.0)
                w = w / jnp.sum(w, axis=1, keepdims=True)
                w_heads.append(w.astype(jnp.bfloat16))

            @pl.when(has_left)
            def _(b=b):
                halo_rdma(1, vsend, vbuf, b, 0, left).wait_recv()

            @pl.when(has_right)
            def _(b=b):
                halo_rdma(1, vsend, vbuf, b, 1, right).wait_recv()

            v_full = jnp.concatenate(
                [vbuf[0, b], vsend[b], vbuf[1, b]], axis=0)
            ctx_b = jnp.concatenate(
                [jnp.dot(w_heads[h], v_full[:, h * DH:(h + 1) * DH],
                         preferred_element_type=jnp.float32)
                 for h in range(HQ)], axis=1)
            out_vmem[b] = jnp.dot(ctx_b, wo_ref[...],
                                  preferred_element_type=jnp.float32)
            pltpu.make_async_copy(out_vmem.at[b], out_ref.at[b],
                                  out_sems.at[b]).start()

        for b in range(B):
            @pl.when(has_right)
            def _(b=b):
                halo_rdma(0, ksend, kbuf, b, 0, right).wait_send()
                halo_rdma(1, vsend, vbuf, b, 0, right).wait_send()

            @pl.when(has_left)
            def _(b=b):
                halo_rdma(0, ksend, kbuf, b, 1, left).wait_send()
                halo_rdma(1, vsend, vbuf, b, 1, left).wait_send()

        for b in range(B):
            pltpu.make_async_copy(out_vmem.at[b], out_ref.at[b],
                                  out_sems.at[b]).wait()

    return pl.pallas_call(
        body,
        out_shape=jax.ShapeDtypeStruct((B, S, d_model), jnp.float32),
        in_specs=[pl.BlockSpec(memory_space=pltpu.VMEM)] * 5,
        out_specs=pl.BlockSpec(memory_space=pltpu.MemorySpace.HBM),
        scratch_shapes=[
            pltpu.VMEM((B, S, HD), jnp.bfloat16),
            pltpu.VMEM((B, S, HD), jnp.bfloat16),
            pltpu.VMEM((2, B, S, HD), jnp.bfloat16),
            pltpu.VMEM((2, B, S, HD), jnp.bfloat16),
            pltpu.VMEM((B, S, 512), jnp.float32),
            pltpu.SemaphoreType.DMA((8,)),
            pltpu.SemaphoreType.DMA((8,)),
            pltpu.SemaphoreType.DMA((2,)),
        ],
        compiler_params=pltpu.CompilerParams(collective_id=0),
    )(x, Wq, K_ext, V_ext, Wo)


# device time: 11467 ns/iter; 1.2314x vs baseline; 1.0116x over previous
import jax
import jax.numpy as jnp
from jax import lax
from jax.experimental import pallas as pl
from jax.experimental.pallas import tpu as pltpu

N_DEV = 16
B = 2
S = 128
HQ = 4
DH = 64
HD = HQ * DH
NEG = -1e9


def kernel(x, Wq, K_ext, V_ext, Wo):
    d_model = x.shape[-1]

    def body(x_ref, wq_ref, k_ref, v_ref, wo_ref, out_ref,
             ksend, vsend, kbuf, vbuf, out_vmem,
             send_sems, recv_sems, out_sems):
        my = lax.axis_index("i")
        left = my - 1
        right = my + 1
        has_left = my > 0
        has_right = my < N_DEV - 1

        @pl.when(jnp.logical_not(has_left))
        def _():
            kbuf[0] = jnp.zeros((B, S, HD), jnp.bfloat16)
            vbuf[0] = jnp.zeros((B, S, HD), jnp.bfloat16)

        @pl.when(jnp.logical_not(has_right))
        def _():
            kbuf[1] = jnp.zeros((B, S, HD), jnp.bfloat16)
            vbuf[1] = jnp.zeros((B, S, HD), jnp.bfloat16)

        barrier_sem = pltpu.get_barrier_semaphore()

        @pl.when(has_left)
        def _():
            pl.semaphore_signal(barrier_sem, inc=1, device_id=(left,),
                                device_id_type=pl.DeviceIdType.MESH)

        @pl.when(has_right)
        def _():
            pl.semaphore_signal(barrier_sem, inc=1, device_id=(right,),
                                device_id_type=pl.DeviceIdType.MESH)

        for h in range(HQ):
            hs = slice(h * DH, (h + 1) * DH)
            ksend[:, :, hs] = k_ref[:, :, h, :].astype(jnp.bfloat16)
            vsend[:, :, hs] = v_ref[:, :, h, :].astype(jnp.bfloat16)

        n_nbr = has_left.astype(jnp.int32) + has_right.astype(jnp.int32)
        pl.semaphore_wait(barrier_sem, n_nbr)

        def sem_idx(kind, b, slot):
            return (kind * B + b) * 2 + slot

        def halo_rdma(kind, src, buf, b, slot, nbr):
            i = sem_idx(kind, b, slot)
            return pltpu.make_async_remote_copy(
                src_ref=src.at[b],
                dst_ref=buf.at[slot, b],
                send_sem=send_sems.at[i],
                recv_sem=recv_sems.at[i],
                device_id=(nbr,),
                device_id_type=pl.DeviceIdType.MESH,
            )

        for b in range(B):
            @pl.when(has_right)
            def _(b=b):
                halo_rdma(0, ksend, kbuf, b, 0, right).start()

            @pl.when(has_left)
            def _(b=b):
                halo_rdma(0, ksend, kbuf, b, 1, left).start()

            @pl.when(has_right)
            def _(b=b):
                halo_rdma(1, vsend, vbuf, b, 0, right).start()

            @pl.when(has_left)
            def _(b=b):
                halo_rdma(1, vsend, vbuf, b, 1, left).start()

        x2 = x_ref[...].reshape(B * S, x_ref.shape[-1])
        q2 = jnp.dot(x2, wq_ref[...],
                     preferred_element_type=jnp.float32) * 0.125

        qi = lax.broadcasted_iota(jnp.int32, (S, S), 0)
        kj = lax.broadcasted_iota(jnp.int32, (S, S), 1)
        full_mask = jnp.concatenate(
            [jnp.logical_and(qi <= kj, has_left),
             jnp.ones((S, S), jnp.bool_),
             jnp.logical_and(qi >= kj, has_right)], axis=1)

        dn = (((1,), (1,)), ((), ()))
        for b in range(B):
            @pl.when(has_left)
            def _(b=b):
                halo_rdma(0, ksend, kbuf, b, 0, left).wait_recv()

            @pl.when(has_right)
            def _(b=b):
                halo_rdma(0, ksend, kbuf, b, 1, right).wait_recv()

            k_full = jnp.concatenate(
                [kbuf[0, b], ksend[b], kbuf[1, b]], axis=0)
            w_heads = []
            r_heads = []
            for h in range(HQ):
                hs = slice(h * DH, (h + 1) * DH)
                q = q2[b * S:(b + 1) * S, hs].astype(jnp.bfloat16)
                s = lax.dot_general(q, k_full[:, hs], dn,
                                    preferred_element_type=jnp.float32)
                w = jnp.where(full_mask, jnp.exp(s), 0.0)
                r_heads.append(1.0 / jnp.sum(w, axis=1, keepdims=True))
                w_heads.append(w.astype(jnp.bfloat16))

            @pl.when(has_left)
            def _(b=b):
                halo_rdma(1, vsend, vbuf, b, 0, left).wait_recv()

            @pl.when(has_right)
            def _(b=b):
                halo_rdma(1, vsend, vbuf, b, 1, right).wait_recv()

            v_full = jnp.concatenate(
                [vbuf[0, b], vsend[b], vbuf[1, b]], axis=0)
            ctx_b = jnp.concatenate(
                [jnp.dot(w_heads[h], v_full[:, h * DH:(h + 1) * DH],
                         preferred_element_type=jnp.float32) * r_heads[h]
                 for h in range(HQ)], axis=1)
            out_vmem[b] = jnp.dot(ctx_b, wo_ref[...],
                                  preferred_element_type=jnp.float32)
            pltpu.make_async_copy(out_vmem.at[b], out_ref.at[b],
                                  out_sems.at[b]).start()

        for b in range(B):
            @pl.when(has_right)
            def _(b=b):
                halo_rdma(0, ksend, kbuf, b, 0, right).wait_send()
                halo_rdma(1, vsend, vbuf, b, 0, right).wait_send()

            @pl.when(has_left)
            def _(b=b):
                halo_rdma(0, ksend, kbuf, b, 1, left).wait_send()
                halo_rdma(1, vsend, vbuf, b, 1, left).wait_send()

        for b in range(B):
            pltpu.make_async_copy(out_vmem.at[b], out_ref.at[b],
                                  out_sems.at[b]).wait()

    return pl.pallas_call(
        body,
        out_shape=jax.ShapeDtypeStruct((B, S, d_model), jnp.float32),
        in_specs=[pl.BlockSpec(memory_space=pltpu.VMEM)] * 5,
        out_specs=pl.BlockSpec(memory_space=pltpu.MemorySpace.HBM),
        scratch_shapes=[
            pltpu.VMEM((B, S, HD), jnp.bfloat16),
            pltpu.VMEM((B, S, HD), jnp.bfloat16),
            pltpu.VMEM((2, B, S, HD), jnp.bfloat16),
            pltpu.VMEM((2, B, S, HD), jnp.bfloat16),
            pltpu.VMEM((B, S, 512), jnp.float32),
            pltpu.SemaphoreType.DMA((8,)),
            pltpu.SemaphoreType.DMA((8,)),
            pltpu.SemaphoreType.DMA((2,)),
        ],
        compiler_params=pltpu.CompilerParams(collective_id=0),
    )(x, Wq, K_ext, V_ext, Wo)
